# baseline (speedup 1.0000x reference)
import functools

import jax
import jax.numpy as jnp
from jax import lax
from jax.experimental import pallas as pl
from jax.experimental.pallas import tpu as pltpu
from jax.experimental.pallas import tpu_sc as plsc

T, XD, KD, E = 2048, 1024, 512, 8
TM = 256
NT = 24
PP = NT * TM
NC = 2
NW = 32
TPW = T // NW
RPW = PP // NW
RCH = 48


def _gate_body(gate_ref, keys_ref, wg_ref, scores_ref, p1_ref, p2_ref,
               g1_ref, g2_ref, tid_ref):
    q = lax.dot_general(gate_ref[...], wg_ref[...], (((1,), (0,)), ((), ())),
                        preferred_element_type=jnp.float32)
    logits = lax.dot_general(q, keys_ref[...], (((1,), (1,)), ((), ())),
                             preferred_element_type=jnp.float32)
    lane = lax.broadcasted_iota(jnp.int32, (T, E), 1)
    m1 = jnp.max(logits, axis=1, keepdims=True)
    idx1 = jnp.min(jnp.where(logits == m1, lane, E), axis=1, keepdims=True)
    rest = jnp.where(lane == idx1, -jnp.inf, logits)
    m2 = jnp.max(rest, axis=1, keepdims=True)
    idx2 = jnp.min(jnp.where(rest == m2, lane, E), axis=1, keepdims=True)
    ex = jnp.exp(m2 - m1)
    g1 = 1.0 / (1.0 + ex)
    g2 = ex * g1
    scores_ref[...] = (jnp.where(lane == idx1, g1, 0.0)
                       + jnp.where(lane == idx2, g2, 0.0))

    c = (jnp.where(lane == idx1, 1.0, 0.0)
         + jnp.where(lane == idx2, 1.0, 0.0))
    s = c
    sh = 1
    while sh < T:
        s = s + jnp.concatenate(
            [jnp.zeros((sh, E), jnp.float32), s[:T - sh]], axis=0)
        sh *= 2
    excl = s - c
    counts = s[T - 1:T, :].astype(jnp.int32)
    ntiles = (counts + (TM - 1)) >> 8
    tb = ntiles
    for lsh in (1, 2, 4):
        tb = tb + jnp.concatenate(
            [jnp.zeros((1, lsh), jnp.int32), tb[:, :E - lsh]], axis=1)
    offs = ((tb - ntiles) * TM).astype(jnp.float32)

    def pick(idx):
        m = lane == idx
        r = jnp.sum(jnp.where(m, excl, 0.0), axis=1, keepdims=True)
        o = jnp.sum(jnp.where(m, jnp.broadcast_to(offs, (T, E)), 0.0),
                    axis=1, keepdims=True)
        return (r + o).astype(jnp.int32)

    p1_ref[...] = pick(idx1)
    p2_ref[...] = pick(idx2)
    g1_ref[...] = g1
    g2_ref[...] = g2

    ti = lax.broadcasted_iota(jnp.int32, (8, 128), 1)
    acc = jnp.zeros((8, 128), jnp.int32)
    for e in range(E):
        acc = acc + jnp.where(ti >= jnp.broadcast_to(tb[:, e:e + 1], (8, 128)),
                              1, 0)
    tid_ref[...] = jnp.minimum(acc, E - 1)


def _gating(gate_inputs, keys, W_gate):
    return pl.pallas_call(
        _gate_body,
        grid=(1,),
        in_specs=[
            pl.BlockSpec((T, XD), lambda i: (0, 0)),
            pl.BlockSpec((E, KD), lambda i: (0, 0)),
            pl.BlockSpec((XD, KD), lambda i: (0, 0)),
        ],
        out_specs=[
            pl.BlockSpec((T, E), lambda i: (0, 0)),
            pl.BlockSpec((T, 1), lambda i: (0, 0)),
            pl.BlockSpec((T, 1), lambda i: (0, 0)),
            pl.BlockSpec((T, 1), lambda i: (0, 0)),
            pl.BlockSpec((T, 1), lambda i: (0, 0)),
            pl.BlockSpec((8, 128), lambda i: (0, 0)),
        ],
        out_shape=[
            jax.ShapeDtypeStruct((T, E), jnp.float32),
            jax.ShapeDtypeStruct((T, 1), jnp.int32),
            jax.ShapeDtypeStruct((T, 1), jnp.int32),
            jax.ShapeDtypeStruct((T, 1), jnp.float32),
            jax.ShapeDtypeStruct((T, 1), jnp.float32),
            jax.ShapeDtypeStruct((8, 128), jnp.int32),
        ],
    )(gate_inputs, keys, W_gate)


@functools.cache
def _mesh():
    return plsc.VectorSubcoreMesh(core_axis_name="c", subcore_axis_name="s")


def _wid():
    return lax.axis_index("s") * NC + lax.axis_index("c")


def _b1_body(p1_hbm, p2_hbm, g1_hbm, g2_hbm, stok_hbm, gs_hbm,
             pidx, tok, gv, s1, s2):
    tbase = _wid() * TPW
    pltpu.sync_copy(p1_hbm.at[pl.ds(tbase, TPW)], pidx.at[pl.ds(0, TPW)])
    pltpu.sync_copy(p2_hbm.at[pl.ds(tbase, TPW)], pidx.at[pl.ds(TPW, TPW)])
    pltpu.sync_copy(g1_hbm.at[pl.ds(tbase, TPW)], gv.at[pl.ds(0, TPW)])
    pltpu.sync_copy(g2_hbm.at[pl.ds(tbase, TPW)], gv.at[pl.ds(TPW, TPW)])
    for c in range(TPW // 16):
        t16 = lax.iota(jnp.int32, 16) + (tbase + c * 16)
        tok[pl.ds(c * 16, 16)] = t16
        tok[pl.ds(TPW + c * 16, 16)] = t16
    cp1 = pltpu.async_copy(tok, stok_hbm.at[pidx], s1)
    cp2 = pltpu.async_copy(gv, gs_hbm.at[pidx], s2)
    cp1.wait()
    cp2.wait()


@functools.cache
def _b1():
    return functools.partial(
        pl.kernel, mesh=_mesh(),
        out_type=[jax.ShapeDtypeStruct((PP,), jnp.int32),
                  jax.ShapeDtypeStruct((PP,), jnp.float32)],
        scratch_types=[pltpu.VMEM((2 * TPW,), jnp.int32),
                       pltpu.VMEM((2 * TPW,), jnp.int32),
                       pltpu.VMEM((2 * TPW,), jnp.float32),
                       pltpu.SemaphoreType.DMA,
                       pltpu.SemaphoreType.DMA])(_b1_body)


def _b2_body(stok_hbm, raw_hbm, xs_hbm, idxv, buf0, buf1, s0, s1):
    base = _wid() * RPW
    pltpu.sync_copy(stok_hbm.at[pl.ds(base, RPW)], idxv)
    for c in range(RPW // 16):
        v = idxv[pl.ds(c * 16, 16)]
        idxv[pl.ds(c * 16, 16)] = jnp.minimum(jnp.maximum(v, 0), T - 1)
    for c in range(RPW // RCH):
        buf = buf0 if c % 2 == 0 else buf1
        sem = s0 if c % 2 == 0 else s1
        pltpu.async_copy(
            raw_hbm.at[idxv.at[pl.ds(c * RCH, RCH)]], buf, sem).wait()
        pltpu.sync_copy(buf, xs_hbm.at[pl.ds(base + c * RCH, RCH)])


@functools.cache
def _b2():
    return functools.partial(
        pl.kernel, mesh=_mesh(),
        out_type=jax.ShapeDtypeStruct((PP, XD), jnp.float32),
        scratch_types=[pltpu.VMEM((RPW,), jnp.int32),
                       pltpu.VMEM((RCH, XD), jnp.float32),
                       pltpu.VMEM((RCH, XD), jnp.float32),
                       pltpu.SemaphoreType.DMA,
                       pltpu.SemaphoreType.DMA])(_b2_body)


def _group_body(tid_ref, x_ref, gs_ref, we_ref, y_ref):
    xs = x_ref[...] * gs_ref[...]
    y_ref[...] = lax.dot_general(xs, we_ref[0], (((1,), (0,)), ((), ())),
                                 preferred_element_type=jnp.float32)


def _grouped_matmul(tid_arr, xs, gs16, W_expert):
    grid_spec = pltpu.PrefetchScalarGridSpec(
        num_scalar_prefetch=1,
        grid=(NT,),
        in_specs=[
            pl.BlockSpec((TM, XD), lambda i, tid: (i, 0)),
            pl.BlockSpec((TM, 1), lambda i, tid: (i, 0)),
            pl.BlockSpec((1, XD, XD), lambda i, tid: (tid[i], 0, 0)),
        ],
        out_specs=pl.BlockSpec((TM, XD), lambda i, tid: (i, 0)),
    )
    return pl.pallas_call(
        _group_body,
        grid_spec=grid_spec,
        out_shape=jax.ShapeDtypeStruct((PP, XD), jnp.float32),
    )(tid_arr, xs, gs16, W_expert)


def _d_body(p1_hbm, p2_hbm, y_hbm, out_hbm, p1v, p2v, bufa, bufb, sa, sb):
    tbase = _wid() * TPW
    pltpu.sync_copy(p1_hbm.at[pl.ds(tbase, TPW)], p1v)
    pltpu.sync_copy(p2_hbm.at[pl.ds(tbase, TPW)], p2v)
    for c in range(2):
        cpa = pltpu.async_copy(y_hbm.at[p1v.at[pl.ds(c * 32, 32)]], bufa, sa)
        cpb = pltpu.async_copy(y_hbm.at[p2v.at[pl.ds(c * 32, 32)]], bufb, sb)
        cpa.wait()
        cpb.wait()

        def row_body(r, carry):
            def col_body(j, carry2):
                off = pl.multiple_of(j * 16, 16)
                bufa[r, pl.ds(off, 16)] = (bufa[r, pl.ds(off, 16)]
                                           + bufb[r, pl.ds(off, 16)])
                return carry2
            return lax.fori_loop(0, XD // 16, col_body, carry)

        lax.fori_loop(0, 32, row_body, 0)
        pltpu.sync_copy(bufa, out_hbm.at[pl.ds(tbase + c * 32, 32)])


@functools.cache
def _d():
    return functools.partial(
        pl.kernel, mesh=_mesh(),
        out_type=jax.ShapeDtypeStruct((T, XD), jnp.float32),
        scratch_types=[pltpu.VMEM((TPW,), jnp.int32),
                       pltpu.VMEM((TPW,), jnp.int32),
                       pltpu.VMEM((32, XD), jnp.float32),
                       pltpu.VMEM((32, XD), jnp.float32),
                       pltpu.SemaphoreType.DMA,
                       pltpu.SemaphoreType.DMA])(_d_body)


def kernel(gate_inputs, raw_inputs, keys, W_gate, W_expert):
    scores, p1c, p2c, g1c, g2c, tid = _gating(gate_inputs, keys, W_gate)
    p1f, p2f = p1c.reshape(-1), p2c.reshape(-1)
    stok, gsort = _b1()(p1f, p2f, g1c.reshape(-1), g2c.reshape(-1))
    xs = _b2()(stok, raw_inputs)
    y = _grouped_matmul(tid[0, :NT], xs, gsort.reshape(PP, 1), W_expert)
    out = _d()(p1f, p2f, y)
    return out, scores


# speedup vs baseline: 3.6913x; 3.6913x over previous
import jax
import jax.numpy as jnp
from jax import lax
from jax.experimental import pallas as pl

T, XD, KD, E = 2048, 1024, 512, 8
TT = 1024


def _body(gate_ref, raw_ref, keys_ref, wg_ref, we_ref, out_ref, scores_ref):
    j = pl.program_id(1)

    @pl.when(j == 0)
    def _gate():
        q = lax.dot_general(
            gate_ref[...], wg_ref[...], (((1,), (0,)), ((), ())),
            preferred_element_type=jnp.float32)
        logits = lax.dot_general(
            q, keys_ref[...], (((1,), (1,)), ((), ())),
            preferred_element_type=jnp.float32)
        lane = lax.broadcasted_iota(jnp.int32, (TT, E), 1)
        m1 = jnp.max(logits, axis=1, keepdims=True)
        idx1 = jnp.min(jnp.where(logits == m1, lane, E), axis=1, keepdims=True)
        rest = jnp.where(lane == idx1, -jnp.inf, logits)
        m2 = jnp.max(rest, axis=1, keepdims=True)
        idx2 = jnp.min(jnp.where(rest == m2, lane, E), axis=1, keepdims=True)
        ex = jnp.exp(m2 - m1)
        g1 = 1.0 / (1.0 + ex)
        g2 = ex * g1
        scores_ref[...] = (jnp.where(lane == idx1, g1, 0.0)
                           + jnp.where(lane == idx2, g2, 0.0))

    lane = lax.broadcasted_iota(jnp.int32, (TT, E), 1)
    col = jnp.sum(jnp.where(lane == j, scores_ref[...], 0.0),
                  axis=1, keepdims=True)
    contrib = col * lax.dot_general(
        raw_ref[...], we_ref[0], (((1,), (0,)), ((), ())),
        preferred_element_type=jnp.float32)

    @pl.when(j == 0)
    def _init():
        out_ref[...] = contrib

    @pl.when(j > 0)
    def _acc():
        out_ref[...] += contrib


def kernel(gate_inputs, raw_inputs, keys, W_gate, W_expert):
    out, scores = pl.pallas_call(
        _body,
        grid=(T // TT, E),
        in_specs=[
            pl.BlockSpec((TT, XD), lambda i, j: (i, 0)),
            pl.BlockSpec((TT, XD), lambda i, j: (i, 0)),
            pl.BlockSpec((E, KD), lambda i, j: (0, 0)),
            pl.BlockSpec((XD, KD), lambda i, j: (0, 0)),
            pl.BlockSpec((1, XD, XD), lambda i, j: (j, 0, 0)),
        ],
        out_specs=[
            pl.BlockSpec((TT, XD), lambda i, j: (i, 0)),
            pl.BlockSpec((TT, E), lambda i, j: (i, 0)),
        ],
        out_shape=[
            jax.ShapeDtypeStruct((T, XD), jnp.float32),
            jax.ShapeDtypeStruct((T, E), jnp.float32),
        ],
    )(gate_inputs, raw_inputs, keys, W_gate, W_expert)
    return out, scores
